# Initial kernel scaffold; baseline (speedup 1.0000x reference)
import jax
import jax.numpy as jnp
from jax.experimental import pallas as pl


def kernel(s, v, pos, edge_index, W_edge1, b_edge1, W_edge2, b_edge2, W_vec, W_v0, W_s1, b_s1, W_s2, b_s2, W_v1):
    raise NotImplementedError("write your pallas kernel here")


# speedup vs baseline: 6.6064x; 6.6064x over previous
import functools

import jax
import jax.numpy as jnp
import numpy as np
from jax import lax
from jax.experimental import pallas as pl
from jax.experimental.pallas import tpu as pltpu

SI = 128
VI = 32
K = 20
CUTOFF = 5.0

TDST_W = 144
TSRC_W = 112


def _silu(x):
    return x * jax.nn.sigmoid(x)


def _prep_body(s_ref, v_ref, pos_ref, wvec_t_ref, tdst_ref, tsrc_ref):
    s = s_ref[...]
    vflat = v_ref[...]
    pos = pos_ref[...]
    wv = wvec_t_ref[...]
    b = s.shape[0]
    pad = jnp.zeros((b, 13), jnp.float32)
    vw = [jnp.dot(vflat[:, c * VI:(c + 1) * VI], wv,
                  preferred_element_type=jnp.float32) for c in range(3)]
    tdst_ref[...] = jnp.concatenate([s, pos, pad], axis=1)
    tsrc_ref[...] = jnp.concatenate(vw + [pos, pad], axis=1)


def _prep_call(s, vflat, pos, wvec_t, n):
    bn = 1000
    grid = (n // bn,)
    return pl.pallas_call(
        _prep_body,
        grid=grid,
        in_specs=[
            pl.BlockSpec((bn, SI), lambda i: (i, 0)),
            pl.BlockSpec((bn, 3 * VI), lambda i: (i, 0)),
            pl.BlockSpec((bn, 3), lambda i: (i, 0)),
            pl.BlockSpec((VI, VI), lambda i: (0, 0)),
        ],
        out_specs=[
            pl.BlockSpec((bn, TDST_W), lambda i: (i, 0)),
            pl.BlockSpec((bn, TSRC_W), lambda i: (i, 0)),
        ],
        out_shape=[
            jax.ShapeDtypeStruct((n, TDST_W), jnp.float32),
            jax.ShapeDtypeStruct((n, TSRC_W), jnp.float32),
        ],
    )(s, vflat, pos, wvec_t)


def _edge_body(tdst_ref, tsrc_ref, sj_ref, w1_t_ref, b1_ref, w2_t_ref,
               b2_ref, ms_ref, mvd_ref):
    tdst = tdst_ref[...]
    tsrc = tsrc_ref[...]
    s_j = sj_ref[...]
    b = tdst.shape[0]

    s_i = tdst[:, :SI]
    pos_i = tdst[:, SI:SI + 3]
    pos_j = tsrc[:, 3 * VI:3 * VI + 3]
    rel = pos_i - pos_j
    d2 = jnp.sum(rel * rel, axis=1, keepdims=True)
    d = jnp.sqrt(d2 + 1e-12)
    inv_d = 1.0 / d
    r_unit = rel * inv_d

    ks = lax.broadcasted_iota(jnp.int32, (1, K), 1).astype(jnp.float32) + 1.0
    t = (d * (1.0 / CUTOFF)) * ks
    r = jnp.round(t)
    w = t - r
    half_par = r * 0.5
    sign = 1.0 - 4.0 * (half_par - jnp.floor(half_par))
    de = sign * jnp.sin(np.pi * w) * inv_d * np.sqrt(2.0 / CUTOFF)
    rs = d * (1.0 / CUTOFF)
    rs2 = rs * rs
    rs3 = rs2 * rs
    rs6 = rs3 * rs3
    rs7 = rs6 * rs
    rs8 = rs7 * rs
    dc = (1.0 - 28.0 * rs6 + 48.0 * rs7 - 21.0 * rs8) * (rs < 1.0)
    de = de * dc

    w1 = w1_t_ref[...]
    hpre = (jnp.dot(s_i, w1[:SI], preferred_element_type=jnp.float32)
            + jnp.dot(s_j, w1[SI:2 * SI], preferred_element_type=jnp.float32)
            + jnp.dot(de, w1[2 * SI:], preferred_element_type=jnp.float32)
            + b1_ref[...])
    h = _silu(hpre)
    a = jnp.dot(h, w2_t_ref[...], preferred_element_type=jnp.float32) + b2_ref[...]
    a_s = a[:, :SI]
    w0 = a[:, SI:SI + VI]
    w1g = a[:, SI + VI:SI + 2 * VI]
    w2g = a[:, SI + 2 * VI:]

    ms_ref[...] = a_s * s_j

    vj = [tsrc[:, c * VI:(c + 1) * VI] for c in range(3)]
    r = [r_unit[:, c:c + 1] for c in range(3)]
    cross = [vj[1] * r[2] - vj[2] * r[1],
             vj[2] * r[0] - vj[0] * r[2],
             vj[0] * r[1] - vj[1] * r[0]]
    mv = [w0 * vj[c] + w1g * r[c] + w2g * cross[c] for c in range(3)]
    ones = jnp.ones((b, 1), jnp.float32)
    zpad = jnp.zeros((b, VI - 1), jnp.float32)
    mvd_ref[...] = jnp.concatenate(mv + [ones, zpad], axis=1)


def _edge_call(tdg, tsg, sjg, w1_t, b1, w2_t, b2, e):
    be = 1280
    assert e % be == 0
    grid = (e // be,)
    return pl.pallas_call(
        _edge_body,
        grid=grid,
        in_specs=[
            pl.BlockSpec((be, TDST_W), lambda i: (i, 0)),
            pl.BlockSpec((be, TSRC_W), lambda i: (i, 0)),
            pl.BlockSpec((be, SI), lambda i: (i, 0)),
            pl.BlockSpec((2 * SI + K, SI), lambda i: (0, 0)),
            pl.BlockSpec((1, SI), lambda i: (0, 0)),
            pl.BlockSpec((SI, SI + 3 * VI), lambda i: (0, 0)),
            pl.BlockSpec((1, SI + 3 * VI), lambda i: (0, 0)),
        ],
        out_specs=[
            pl.BlockSpec((be, SI), lambda i: (i, 0)),
            pl.BlockSpec((be, SI), lambda i: (i, 0)),
        ],
        out_shape=[
            jax.ShapeDtypeStruct((e, SI), jnp.float32),
            jax.ShapeDtypeStruct((e, SI), jnp.float32),
        ],
    )(tdg, tsg, sjg, w1_t, b1, w2_t, b2)


def _node_body(msagg_ref, mvd_ref, s_ref, v_ref, wv0_t_ref, ws1_t_ref,
               bs1_ref, ws2_t_ref, bs2_ref, wv1_t_ref, sout_ref, vout_ref):
    ms_agg = msagg_ref[...]
    mvd = mvd_ref[...]
    s = s_ref[...]
    vflat = v_ref[...]

    deg = mvd[:, 3 * VI:3 * VI + 1]
    inv_deg = 1.0 / jnp.maximum(deg, 1.0)
    wv0 = wv0_t_ref[...]
    vv = [jnp.dot(mvd[:, c * VI:(c + 1) * VI] * inv_deg, wv0,
                  preferred_element_type=jnp.float32) for c in range(3)]
    vn2 = sum(vv[c][:, :VI] * vv[c][:, :VI] for c in range(3))
    vnorm = jnp.sqrt(jnp.maximum(vn2, 1e-6))
    scat = jnp.concatenate([ms_agg, vnorm], axis=1)
    hh = _silu(jnp.dot(scat, ws1_t_ref[...], preferred_element_type=jnp.float32)
               + bs1_ref[...])
    o = jnp.dot(hh, ws2_t_ref[...], preferred_element_type=jnp.float32) + bs2_ref[...]
    gate = o[:, :VI]
    s_upd = o[:, VI:]
    wv1 = wv1_t_ref[...]
    vupd = [jnp.dot(gate * vv[c][:, VI:], wv1,
                    preferred_element_type=jnp.float32) for c in range(3)]
    sout_ref[...] = s + s_upd
    vout_ref[...] = vflat + jnp.concatenate(vupd, axis=1)


def _node_call(ms_agg, mvd_agg, s, vflat, wv0_t, ws1_t, bs1, ws2_t, bs2,
               wv1_t, n):
    bn = 1000
    grid = (n // bn,)
    return pl.pallas_call(
        _node_body,
        grid=grid,
        in_specs=[
            pl.BlockSpec((bn, SI), lambda i: (i, 0)),
            pl.BlockSpec((bn, SI), lambda i: (i, 0)),
            pl.BlockSpec((bn, SI), lambda i: (i, 0)),
            pl.BlockSpec((bn, 3 * VI), lambda i: (i, 0)),
            pl.BlockSpec((VI, 2 * VI), lambda i: (0, 0)),
            pl.BlockSpec((VI + SI, SI), lambda i: (0, 0)),
            pl.BlockSpec((1, SI), lambda i: (0, 0)),
            pl.BlockSpec((SI, VI + SI), lambda i: (0, 0)),
            pl.BlockSpec((1, VI + SI), lambda i: (0, 0)),
            pl.BlockSpec((VI, VI), lambda i: (0, 0)),
        ],
        out_specs=[
            pl.BlockSpec((bn, SI), lambda i: (i, 0)),
            pl.BlockSpec((bn, 3 * VI), lambda i: (i, 0)),
        ],
        out_shape=[
            jax.ShapeDtypeStruct((n, SI), jnp.float32),
            jax.ShapeDtypeStruct((n, 3 * VI), jnp.float32),
        ],
    )(ms_agg, mvd_agg, s, vflat, wv0_t, ws1_t, bs1, ws2_t, bs2, wv1_t)


def kernel(s, v, pos, edge_index, W_edge1, b_edge1, W_edge2, b_edge2,
           W_vec, W_v0, W_s1, b_s1, W_s2, b_s2, W_v1):
    n = s.shape[0]
    e = edge_index.shape[1]
    vflat = v.reshape(n, 3 * VI)
    src = edge_index[0]
    dst = edge_index[1]

    tdst, tsrc = _prep_call(s, vflat, pos, W_vec.T, n)

    tdg = jnp.take(tdst, dst, axis=0)
    tsg = jnp.take(tsrc, src, axis=0)
    sjg = jnp.take(s, src, axis=0)

    ms, mvd = _edge_call(tdg, tsg, sjg, W_edge1.T, b_edge1[None, :],
                         W_edge2.T, b_edge2[None, :], e)

    ms_agg = jax.ops.segment_sum(ms, dst, num_segments=n)
    mvd_agg = jax.ops.segment_sum(mvd, dst, num_segments=n)

    s_out, v_out = _node_call(ms_agg, mvd_agg, s, vflat, W_v0.T, W_s1.T,
                              b_s1[None, :], W_s2.T, b_s2[None, :], W_v1.T, n)
    return (s_out, v_out.reshape(n, 3, VI))


# speedup vs baseline: 10.5147x; 1.5916x over previous
import functools

import jax
import jax.numpy as jnp
import numpy as np
from jax import lax
from jax.experimental import pallas as pl
from jax.experimental.pallas import tpu as pltpu
from jax.experimental.pallas import tpu_sc as plsc

SI = 128
VI = 32
K = 20
CUTOFF = 5.0

TDST_W = 256
TSRC_W = 256


def _silu(x):
    return x * jax.nn.sigmoid(x)


def _prep_body(s_ref, v_ref, pos_ref, wvec_t_ref, tdst_ref, tsrc_ref):
    s = s_ref[...]
    vflat = v_ref[...]
    pos = pos_ref[...]
    wv = wvec_t_ref[...]
    b = s.shape[0]
    vw = [jnp.dot(vflat[:, c * VI:(c + 1) * VI], wv,
                  preferred_element_type=jnp.float32) for c in range(3)]
    pad_d = jnp.zeros((b, TDST_W - SI - 3), jnp.float32)
    pad_s = jnp.zeros((b, TSRC_W - 2 * SI - 3 + VI), jnp.float32)
    tdst_ref[...] = jnp.concatenate([s, pos, pad_d], axis=1)
    tsrc_ref[...] = jnp.concatenate([s] + vw + [pos, pad_s], axis=1)


def _prep_call(s, vflat, pos, wvec_t, n):
    bn = 1000
    grid = (n // bn,)
    return pl.pallas_call(
        _prep_body,
        grid=grid,
        in_specs=[
            pl.BlockSpec((bn, SI), lambda i: (i, 0)),
            pl.BlockSpec((bn, 3 * VI), lambda i: (i, 0)),
            pl.BlockSpec((bn, 3), lambda i: (i, 0)),
            pl.BlockSpec((VI, VI), lambda i: (0, 0)),
        ],
        out_specs=[
            pl.BlockSpec((bn, TDST_W), lambda i: (i, 0)),
            pl.BlockSpec((bn, TSRC_W), lambda i: (i, 0)),
        ],
        out_shape=[
            jax.ShapeDtypeStruct((n, TDST_W), jnp.float32),
            jax.ShapeDtypeStruct((n, TSRC_W), jnp.float32),
        ],
    )(s, vflat, pos, wvec_t)


GCH = 80


def _gather_call(tdst, tsrc, src_idx, dst_idx, e):
    n_workers = 32
    per = e // n_workers
    assert per % GCH == 0
    mesh = plsc.VectorSubcoreMesh(core_axis_name="c", subcore_axis_name="s")

    @functools.partial(
        pl.kernel,
        mesh=mesh,
        out_type=[
            jax.ShapeDtypeStruct((e, TDST_W), jnp.float32),
            jax.ShapeDtypeStruct((e, TSRC_W), jnp.float32),
        ],
        scratch_types=[
            pltpu.VMEM((GCH,), jnp.int32),
            pltpu.VMEM((GCH,), jnp.int32),
            pltpu.VMEM((GCH, TDST_W), jnp.float32),
            pltpu.VMEM((GCH, TSRC_W), jnp.float32),
            pltpu.SemaphoreType.DMA,
            pltpu.SemaphoreType.DMA,
        ],
    )
    def gather_k(tdst_hbm, tsrc_hbm, src_hbm, dst_hbm,
                 otd_hbm, ots_hbm,
                 dsti_v, srci_v, btd_v, bts_v, sem1, sem2):
        wid = lax.axis_index("s") * 2 + lax.axis_index("c")
        tile_base = wid * per

        def body(i, carry):
            base = tile_base + i * GCH
            pltpu.sync_copy(dst_hbm.at[pl.ds(base, GCH)], dsti_v)
            pltpu.sync_copy(src_hbm.at[pl.ds(base, GCH)], srci_v)
            c1 = pltpu.async_copy(tdst_hbm.at[dsti_v], btd_v, sem1)
            c2 = pltpu.async_copy(tsrc_hbm.at[srci_v], bts_v, sem2)
            c1.wait()
            c2.wait()
            pltpu.sync_copy(btd_v, otd_hbm.at[pl.ds(base, GCH)])
            pltpu.sync_copy(bts_v, ots_hbm.at[pl.ds(base, GCH)])
            return carry

        lax.fori_loop(0, per // GCH, body, 0)

    return gather_k(tdst, tsrc, src_idx, dst_idx)


def _edge_body(tdst_ref, tsrc_ref, w1_t_ref, b1_ref, w2_t_ref,
               b2_ref, ms_ref, mvd_ref):
    tdst = tdst_ref[...]
    tsrc = tsrc_ref[...]
    s_j = tsrc[:, :SI]
    b = tdst.shape[0]

    s_i = tdst[:, :SI]
    pos_i = tdst[:, SI:SI + 3]
    pos_j = tsrc[:, SI + 3 * VI:SI + 3 * VI + 3]
    rel = pos_i - pos_j
    d2 = jnp.sum(rel * rel, axis=1, keepdims=True)
    d = jnp.sqrt(d2 + 1e-12)
    inv_d = 1.0 / d
    r_unit = rel * inv_d

    ks = lax.broadcasted_iota(jnp.int32, (1, K), 1).astype(jnp.float32) + 1.0
    t = (d * (1.0 / CUTOFF)) * ks
    r = jnp.round(t)
    w = t - r
    half_par = r * 0.5
    sign = 1.0 - 4.0 * (half_par - jnp.floor(half_par))
    de = sign * jnp.sin(np.pi * w) * inv_d * np.sqrt(2.0 / CUTOFF)
    rs = d * (1.0 / CUTOFF)
    rs2 = rs * rs
    rs3 = rs2 * rs
    rs6 = rs3 * rs3
    rs7 = rs6 * rs
    rs8 = rs7 * rs
    dc = (1.0 - 28.0 * rs6 + 48.0 * rs7 - 21.0 * rs8) * (rs < 1.0)
    de = de * dc

    w1 = w1_t_ref[...]
    hpre = (jnp.dot(s_i, w1[:SI], preferred_element_type=jnp.float32)
            + jnp.dot(s_j, w1[SI:2 * SI], preferred_element_type=jnp.float32)
            + jnp.dot(de, w1[2 * SI:], preferred_element_type=jnp.float32)
            + b1_ref[...])
    h = _silu(hpre)
    a = jnp.dot(h, w2_t_ref[...], preferred_element_type=jnp.float32) + b2_ref[...]
    a_s = a[:, :SI]
    w0 = a[:, SI:SI + VI]
    w1g = a[:, SI + VI:SI + 2 * VI]
    w2g = a[:, SI + 2 * VI:]

    ms_ref[...] = a_s * s_j

    vj = [tsrc[:, SI + c * VI:SI + (c + 1) * VI] for c in range(3)]
    r = [r_unit[:, c:c + 1] for c in range(3)]
    cross = [vj[1] * r[2] - vj[2] * r[1],
             vj[2] * r[0] - vj[0] * r[2],
             vj[0] * r[1] - vj[1] * r[0]]
    mv = [w0 * vj[c] + w1g * r[c] + w2g * cross[c] for c in range(3)]
    ones = jnp.ones((b, 1), jnp.float32)
    zpad = jnp.zeros((b, VI - 1), jnp.float32)
    mvd_ref[...] = jnp.concatenate(mv + [ones, zpad], axis=1)


def _edge_call(tdg, tsg, w1_t, b1, w2_t, b2, e):
    be = 1280
    assert e % be == 0
    grid = (e // be,)
    return pl.pallas_call(
        _edge_body,
        grid=grid,
        in_specs=[
            pl.BlockSpec((be, TDST_W), lambda i: (i, 0)),
            pl.BlockSpec((be, TSRC_W), lambda i: (i, 0)),
            pl.BlockSpec((2 * SI + K, SI), lambda i: (0, 0)),
            pl.BlockSpec((1, SI), lambda i: (0, 0)),
            pl.BlockSpec((SI, SI + 3 * VI), lambda i: (0, 0)),
            pl.BlockSpec((1, SI + 3 * VI), lambda i: (0, 0)),
        ],
        out_specs=[
            pl.BlockSpec((be, SI), lambda i: (i, 0)),
            pl.BlockSpec((be, SI), lambda i: (i, 0)),
        ],
        out_shape=[
            jax.ShapeDtypeStruct((e, SI), jnp.float32),
            jax.ShapeDtypeStruct((e, SI), jnp.float32),
        ],
    )(tdg, tsg, w1_t, b1, w2_t, b2)


def _node_body(msagg_ref, mvd_ref, s_ref, v_ref, wv0_t_ref, ws1_t_ref,
               bs1_ref, ws2_t_ref, bs2_ref, wv1_t_ref, sout_ref, vout_ref):
    ms_agg = msagg_ref[...]
    mvd = mvd_ref[...]
    s = s_ref[...]
    vflat = v_ref[...]

    deg = mvd[:, 3 * VI:3 * VI + 1]
    inv_deg = 1.0 / jnp.maximum(deg, 1.0)
    wv0 = wv0_t_ref[...]
    vv = [jnp.dot(mvd[:, c * VI:(c + 1) * VI] * inv_deg, wv0,
                  preferred_element_type=jnp.float32) for c in range(3)]
    vn2 = sum(vv[c][:, :VI] * vv[c][:, :VI] for c in range(3))
    vnorm = jnp.sqrt(jnp.maximum(vn2, 1e-6))
    scat = jnp.concatenate([ms_agg, vnorm], axis=1)
    hh = _silu(jnp.dot(scat, ws1_t_ref[...], preferred_element_type=jnp.float32)
               + bs1_ref[...])
    o = jnp.dot(hh, ws2_t_ref[...], preferred_element_type=jnp.float32) + bs2_ref[...]
    gate = o[:, :VI]
    s_upd = o[:, VI:]
    wv1 = wv1_t_ref[...]
    vupd = [jnp.dot(gate * vv[c][:, VI:], wv1,
                    preferred_element_type=jnp.float32) for c in range(3)]
    sout_ref[...] = s + s_upd
    vout_ref[...] = vflat + jnp.concatenate(vupd, axis=1)


def _node_call(ms_agg, mvd_agg, s, vflat, wv0_t, ws1_t, bs1, ws2_t, bs2,
               wv1_t, n):
    bn = 1000
    grid = (n // bn,)
    return pl.pallas_call(
        _node_body,
        grid=grid,
        in_specs=[
            pl.BlockSpec((bn, SI), lambda i: (i, 0)),
            pl.BlockSpec((bn, SI), lambda i: (i, 0)),
            pl.BlockSpec((bn, SI), lambda i: (i, 0)),
            pl.BlockSpec((bn, 3 * VI), lambda i: (i, 0)),
            pl.BlockSpec((VI, 2 * VI), lambda i: (0, 0)),
            pl.BlockSpec((VI + SI, SI), lambda i: (0, 0)),
            pl.BlockSpec((1, SI), lambda i: (0, 0)),
            pl.BlockSpec((SI, VI + SI), lambda i: (0, 0)),
            pl.BlockSpec((1, VI + SI), lambda i: (0, 0)),
            pl.BlockSpec((VI, VI), lambda i: (0, 0)),
        ],
        out_specs=[
            pl.BlockSpec((bn, SI), lambda i: (i, 0)),
            pl.BlockSpec((bn, 3 * VI), lambda i: (i, 0)),
        ],
        out_shape=[
            jax.ShapeDtypeStruct((n, SI), jnp.float32),
            jax.ShapeDtypeStruct((n, 3 * VI), jnp.float32),
        ],
    )(ms_agg, mvd_agg, s, vflat, wv0_t, ws1_t, bs1, ws2_t, bs2, wv1_t)


def kernel(s, v, pos, edge_index, W_edge1, b_edge1, W_edge2, b_edge2,
           W_vec, W_v0, W_s1, b_s1, W_s2, b_s2, W_v1):
    n = s.shape[0]
    e = edge_index.shape[1]
    vflat = v.reshape(n, 3 * VI)
    src = edge_index[0]
    dst = edge_index[1]

    tdst, tsrc = _prep_call(s, vflat, pos, W_vec.T, n)

    tdg, tsg = _gather_call(tdst, tsrc, src, dst, e)

    ms, mvd = _edge_call(tdg, tsg, W_edge1.T, b_edge1[None, :],
                         W_edge2.T, b_edge2[None, :], e)

    ms_agg = jax.ops.segment_sum(ms, dst, num_segments=n)
    mvd_agg = jax.ops.segment_sum(mvd, dst, num_segments=n)

    s_out, v_out = _node_call(ms_agg, mvd_agg, s, vflat, W_v0.T, W_s1.T,
                              b_s1[None, :], W_s2.T, b_s2[None, :], W_v1.T, n)
    return (s_out, v_out.reshape(n, 3, VI))


# speedup vs baseline: 14.0704x; 1.3382x over previous
import functools

import jax
import jax.numpy as jnp
import numpy as np
from jax import lax
from jax.experimental import pallas as pl
from jax.experimental.pallas import tpu as pltpu
from jax.experimental.pallas import tpu_sc as plsc

SI = 128
VI = 32
K = 20
CUTOFF = 5.0

TDST_W = 256
TSRC_W = 256


def _silu(x):
    return x * jax.nn.sigmoid(x)


def _prep_body(s_ref, v_ref, pos_ref, wvec_t_ref, tdst_ref, tsrc_ref):
    s = s_ref[...]
    vflat = v_ref[...]
    pos = pos_ref[...]
    wv = wvec_t_ref[...]
    b = s.shape[0]
    vw = [jnp.dot(vflat[:, c * VI:(c + 1) * VI], wv,
                  preferred_element_type=jnp.float32) for c in range(3)]
    pad_d = jnp.zeros((b, TDST_W - SI - 3), jnp.float32)
    pad_s = jnp.zeros((b, TSRC_W - 2 * SI - 3 + VI), jnp.float32)
    tdst_ref[...] = jnp.concatenate([s, pos, pad_d], axis=1)
    tsrc_ref[...] = jnp.concatenate([s] + vw + [pos, pad_s], axis=1)


def _prep_call(s, vflat, pos, wvec_t, n):
    bn = 1000
    grid = (n // bn,)
    return pl.pallas_call(
        _prep_body,
        grid=grid,
        in_specs=[
            pl.BlockSpec((bn, SI), lambda i: (i, 0)),
            pl.BlockSpec((bn, 3 * VI), lambda i: (i, 0)),
            pl.BlockSpec((bn, 3), lambda i: (i, 0)),
            pl.BlockSpec((VI, VI), lambda i: (0, 0)),
        ],
        out_specs=[
            pl.BlockSpec((bn, TDST_W), lambda i: (i, 0)),
            pl.BlockSpec((bn, TSRC_W), lambda i: (i, 0)),
        ],
        out_shape=[
            jax.ShapeDtypeStruct((n, TDST_W), jnp.float32),
            jax.ShapeDtypeStruct((n, TSRC_W), jnp.float32),
        ],
    )(s, vflat, pos, wvec_t)


GCH = 80


def _gather_call(tdst, tsrc, src_idx, dst_idx, e):
    n_workers = 32
    per = e // n_workers
    assert per % GCH == 0
    mesh = plsc.VectorSubcoreMesh(core_axis_name="c", subcore_axis_name="s")

    @functools.partial(
        pl.kernel,
        mesh=mesh,
        out_type=[
            jax.ShapeDtypeStruct((e, TDST_W), jnp.float32),
            jax.ShapeDtypeStruct((e, TSRC_W), jnp.float32),
        ],
        scratch_types=[
            pltpu.VMEM((GCH,), jnp.int32),
            pltpu.VMEM((GCH,), jnp.int32),
            pltpu.VMEM((GCH, TDST_W), jnp.float32),
            pltpu.VMEM((GCH, TSRC_W), jnp.float32),
            pltpu.SemaphoreType.DMA,
            pltpu.SemaphoreType.DMA,
        ],
    )
    def gather_k(tdst_hbm, tsrc_hbm, src_hbm, dst_hbm,
                 otd_hbm, ots_hbm,
                 dsti_v, srci_v, btd_v, bts_v, sem1, sem2):
        wid = lax.axis_index("s") * 2 + lax.axis_index("c")
        tile_base = wid * per

        def body(i, carry):
            base = tile_base + i * GCH
            pltpu.sync_copy(dst_hbm.at[pl.ds(base, GCH)], dsti_v)
            pltpu.sync_copy(src_hbm.at[pl.ds(base, GCH)], srci_v)
            c1 = pltpu.async_copy(tdst_hbm.at[dsti_v], btd_v, sem1)
            c2 = pltpu.async_copy(tsrc_hbm.at[srci_v], bts_v, sem2)
            c1.wait()
            c2.wait()
            pltpu.sync_copy(btd_v, otd_hbm.at[pl.ds(base, GCH)])
            pltpu.sync_copy(bts_v, ots_hbm.at[pl.ds(base, GCH)])
            return carry

        lax.fori_loop(0, per // GCH, body, 0)

    return gather_k(tdst, tsrc, src_idx, dst_idx)


def _edge_body(tdst_ref, tsrc_ref, w1_t_ref, b1_ref, w2_t_ref,
               b2_ref, ms_ref, mvd_ref):
    tdst = tdst_ref[...]
    tsrc = tsrc_ref[...]
    s_j = tsrc[:, :SI]
    b = tdst.shape[0]

    s_i = tdst[:, :SI]
    pos_i = tdst[:, SI:SI + 3]
    pos_j = tsrc[:, SI + 3 * VI:SI + 3 * VI + 3]
    rel = pos_i - pos_j
    d2 = jnp.sum(rel * rel, axis=1, keepdims=True)
    d = jnp.sqrt(d2 + 1e-12)
    inv_d = 1.0 / d
    r_unit = rel * inv_d

    ks = lax.broadcasted_iota(jnp.int32, (1, K), 1).astype(jnp.float32) + 1.0
    t = (d * (1.0 / CUTOFF)) * ks
    r = jnp.round(t)
    w = t - r
    half_par = r * 0.5
    sign = 1.0 - 4.0 * (half_par - jnp.floor(half_par))
    de = sign * jnp.sin(np.pi * w) * inv_d * np.sqrt(2.0 / CUTOFF)
    rs = d * (1.0 / CUTOFF)
    rs2 = rs * rs
    rs3 = rs2 * rs
    rs6 = rs3 * rs3
    rs7 = rs6 * rs
    rs8 = rs7 * rs
    dc = (1.0 - 28.0 * rs6 + 48.0 * rs7 - 21.0 * rs8) * (rs < 1.0)
    de = de * dc

    w1 = w1_t_ref[...]
    hpre = (jnp.dot(s_i, w1[:SI], preferred_element_type=jnp.float32)
            + jnp.dot(s_j, w1[SI:2 * SI], preferred_element_type=jnp.float32)
            + jnp.dot(de, w1[2 * SI:], preferred_element_type=jnp.float32)
            + b1_ref[...])
    h = _silu(hpre)
    a = jnp.dot(h, w2_t_ref[...], preferred_element_type=jnp.float32) + b2_ref[...]
    a_s = a[:, :SI]
    w0 = a[:, SI:SI + VI]
    w1g = a[:, SI + VI:SI + 2 * VI]
    w2g = a[:, SI + 2 * VI:]

    ms_ref[...] = a_s * s_j

    vj = [tsrc[:, SI + c * VI:SI + (c + 1) * VI] for c in range(3)]
    r = [r_unit[:, c:c + 1] for c in range(3)]
    cross = [vj[1] * r[2] - vj[2] * r[1],
             vj[2] * r[0] - vj[0] * r[2],
             vj[0] * r[1] - vj[1] * r[0]]
    mv = [w0 * vj[c] + w1g * r[c] + w2g * cross[c] for c in range(3)]
    ones = jnp.ones((b, 1), jnp.float32)
    zpad = jnp.zeros((b, VI - 1), jnp.float32)
    mvd_ref[...] = jnp.concatenate(mv + [ones, zpad], axis=1)


def _edge_call(tdg, tsg, w1_t, b1, w2_t, b2, e):
    be = 1280
    assert e % be == 0
    grid = (e // be,)
    return pl.pallas_call(
        _edge_body,
        grid=grid,
        in_specs=[
            pl.BlockSpec((be, TDST_W), lambda i: (i, 0)),
            pl.BlockSpec((be, TSRC_W), lambda i: (i, 0)),
            pl.BlockSpec((2 * SI + K, SI), lambda i: (0, 0)),
            pl.BlockSpec((1, SI), lambda i: (0, 0)),
            pl.BlockSpec((SI, SI + 3 * VI), lambda i: (0, 0)),
            pl.BlockSpec((1, SI + 3 * VI), lambda i: (0, 0)),
        ],
        out_specs=[
            pl.BlockSpec((be, SI), lambda i: (i, 0)),
            pl.BlockSpec((be, SI), lambda i: (i, 0)),
        ],
        out_shape=[
            jax.ShapeDtypeStruct((e, SI), jnp.float32),
            jax.ShapeDtypeStruct((e, SI), jnp.float32),
        ],
    )(tdg, tsg, w1_t, b1, w2_t, b2)


SCH = 80


def _scatter_call(ms, mvd, dst_idx, zeros_n, n, e):
    n_tiles = 16
    per = e // n_tiles
    assert per % SCH == 0
    rows_per = (n // n_tiles) & ~7
    tail_base = n_tiles * rows_per
    tail = n - tail_base
    mesh = plsc.VectorSubcoreMesh(core_axis_name="c", subcore_axis_name="s")

    @functools.partial(
        pl.kernel,
        mesh=mesh,
        out_type=[
            jax.ShapeDtypeStruct((n, SI), jnp.float32),
            jax.ShapeDtypeStruct((n, SI), jnp.float32),
        ],
        scratch_types=[
            pltpu.VMEM((SCH,), jnp.int32),
            pltpu.VMEM((SCH, SI), jnp.float32),
            pltpu.VMEM_SHARED((n, SI), jnp.float32),
        ],
    )
    def scatter_k(ms_hbm, mvd_hbm, dst_hbm, zeros_hbm, oms_hbm, omvd_hbm,
                  idx_v, row_v, acc_sh):
        cid = lax.axis_index("c")
        sid = lax.axis_index("s")
        pltpu.sync_copy(zeros_hbm.at[pl.ds(sid * rows_per, rows_per)],
                        acc_sh.at[pl.ds(sid * rows_per, rows_per)])

        @pl.when(sid == 0)
        def _():
            pltpu.sync_copy(zeros_hbm.at[pl.ds(tail_base, tail)],
                            acc_sh.at[pl.ds(tail_base, tail)])

        plsc.subcore_barrier()

        def make_body(src_hbm):
            def body(i, carry):
                base = sid * per + i * SCH
                pltpu.sync_copy(dst_hbm.at[pl.ds(base, SCH)], idx_v)
                pltpu.sync_copy(src_hbm.at[pl.ds(base, SCH)], row_v)
                pltpu.sync_copy(row_v, acc_sh.at[idx_v], add=True)
                return carry
            return body

        @pl.when(cid == 0)
        def _():
            lax.fori_loop(0, per // SCH, make_body(ms_hbm), 0)

        @pl.when(cid == 1)
        def _():
            lax.fori_loop(0, per // SCH, make_body(mvd_hbm), 0)

        plsc.subcore_barrier()

        @pl.when(cid == 0)
        def _():
            pltpu.sync_copy(acc_sh.at[pl.ds(sid * rows_per, rows_per)],
                            oms_hbm.at[pl.ds(sid * rows_per, rows_per)])

            @pl.when(sid == 0)
            def _():
                pltpu.sync_copy(acc_sh.at[pl.ds(tail_base, tail)],
                                oms_hbm.at[pl.ds(tail_base, tail)])

        @pl.when(cid == 1)
        def _():
            pltpu.sync_copy(acc_sh.at[pl.ds(sid * rows_per, rows_per)],
                            omvd_hbm.at[pl.ds(sid * rows_per, rows_per)])

            @pl.when(sid == 0)
            def _():
                pltpu.sync_copy(acc_sh.at[pl.ds(tail_base, tail)],
                                omvd_hbm.at[pl.ds(tail_base, tail)])

    return scatter_k(ms, mvd, dst_idx, zeros_n)


def _node_body(msagg_ref, mvd_ref, s_ref, v_ref, wv0_t_ref, ws1_t_ref,
               bs1_ref, ws2_t_ref, bs2_ref, wv1_t_ref, sout_ref, vout_ref):
    ms_agg = msagg_ref[...]
    mvd = mvd_ref[...]
    s = s_ref[...]
    vflat = v_ref[...]

    deg = mvd[:, 3 * VI:3 * VI + 1]
    inv_deg = 1.0 / jnp.maximum(deg, 1.0)
    wv0 = wv0_t_ref[...]
    vv = [jnp.dot(mvd[:, c * VI:(c + 1) * VI] * inv_deg, wv0,
                  preferred_element_type=jnp.float32) for c in range(3)]
    vn2 = sum(vv[c][:, :VI] * vv[c][:, :VI] for c in range(3))
    vnorm = jnp.sqrt(jnp.maximum(vn2, 1e-6))
    scat = jnp.concatenate([ms_agg, vnorm], axis=1)
    hh = _silu(jnp.dot(scat, ws1_t_ref[...], preferred_element_type=jnp.float32)
               + bs1_ref[...])
    o = jnp.dot(hh, ws2_t_ref[...], preferred_element_type=jnp.float32) + bs2_ref[...]
    gate = o[:, :VI]
    s_upd = o[:, VI:]
    wv1 = wv1_t_ref[...]
    vupd = [jnp.dot(gate * vv[c][:, VI:], wv1,
                    preferred_element_type=jnp.float32) for c in range(3)]
    sout_ref[...] = s + s_upd
    vout_ref[...] = vflat + jnp.concatenate(vupd, axis=1)


def _node_call(ms_agg, mvd_agg, s, vflat, wv0_t, ws1_t, bs1, ws2_t, bs2,
               wv1_t, n):
    bn = 1000
    grid = (n // bn,)
    return pl.pallas_call(
        _node_body,
        grid=grid,
        in_specs=[
            pl.BlockSpec((bn, SI), lambda i: (i, 0)),
            pl.BlockSpec((bn, SI), lambda i: (i, 0)),
            pl.BlockSpec((bn, SI), lambda i: (i, 0)),
            pl.BlockSpec((bn, 3 * VI), lambda i: (i, 0)),
            pl.BlockSpec((VI, 2 * VI), lambda i: (0, 0)),
            pl.BlockSpec((VI + SI, SI), lambda i: (0, 0)),
            pl.BlockSpec((1, SI), lambda i: (0, 0)),
            pl.BlockSpec((SI, VI + SI), lambda i: (0, 0)),
            pl.BlockSpec((1, VI + SI), lambda i: (0, 0)),
            pl.BlockSpec((VI, VI), lambda i: (0, 0)),
        ],
        out_specs=[
            pl.BlockSpec((bn, SI), lambda i: (i, 0)),
            pl.BlockSpec((bn, 3 * VI), lambda i: (i, 0)),
        ],
        out_shape=[
            jax.ShapeDtypeStruct((n, SI), jnp.float32),
            jax.ShapeDtypeStruct((n, 3 * VI), jnp.float32),
        ],
    )(ms_agg, mvd_agg, s, vflat, wv0_t, ws1_t, bs1, ws2_t, bs2, wv1_t)


def kernel(s, v, pos, edge_index, W_edge1, b_edge1, W_edge2, b_edge2,
           W_vec, W_v0, W_s1, b_s1, W_s2, b_s2, W_v1):
    n = s.shape[0]
    e = edge_index.shape[1]
    vflat = v.reshape(n, 3 * VI)
    src = edge_index[0]
    dst = edge_index[1]

    tdst, tsrc = _prep_call(s, vflat, pos, W_vec.T, n)

    tdg, tsg = _gather_call(tdst, tsrc, src, dst, e)

    ms, mvd = _edge_call(tdg, tsg, W_edge1.T, b_edge1[None, :],
                         W_edge2.T, b_edge2[None, :], e)

    zeros_n = jnp.zeros((n, SI), jnp.float32)
    ms_agg, mvd_agg = _scatter_call(ms, mvd, dst, zeros_n, n, e)

    s_out, v_out = _node_call(ms_agg, mvd_agg, s, vflat, W_v0.T, W_s1.T,
                              b_s1[None, :], W_s2.T, b_s2[None, :], W_v1.T, n)
    return (s_out, v_out.reshape(n, 3, VI))


# speedup vs baseline: 16.4297x; 1.1677x over previous
import functools

import jax
import jax.numpy as jnp
import numpy as np
from jax import lax
from jax.experimental import pallas as pl
from jax.experimental.pallas import tpu as pltpu
from jax.experimental.pallas import tpu_sc as plsc

SI = 128
VI = 32
K = 20
CUTOFF = 5.0

TDST_W = 256
TSRC_W = 256


def _silu(x):
    return x * jax.nn.sigmoid(x)


def _prep_body(s_ref, v_ref, pos_ref, wvec_t_ref, tdst_ref, tsrc_ref):
    s = s_ref[...]
    vflat = v_ref[...]
    pos = pos_ref[...]
    wv = wvec_t_ref[...]
    b = s.shape[0]
    vw = [jnp.dot(vflat[:, c * VI:(c + 1) * VI], wv,
                  preferred_element_type=jnp.float32) for c in range(3)]
    pad_d = jnp.zeros((b, TDST_W - SI - 3), jnp.float32)
    pad_s = jnp.zeros((b, TSRC_W - 2 * SI - 3 + VI), jnp.float32)
    tdst_ref[...] = jnp.concatenate([s, pos, pad_d], axis=1)
    tsrc_ref[...] = jnp.concatenate([s] + vw + [pos, pad_s], axis=1)


def _prep_call(s, vflat, pos, wvec_t, n):
    bn = 1000
    grid = (n // bn,)
    return pl.pallas_call(
        _prep_body,
        grid=grid,
        in_specs=[
            pl.BlockSpec((bn, SI), lambda i: (i, 0)),
            pl.BlockSpec((bn, 3 * VI), lambda i: (i, 0)),
            pl.BlockSpec((bn, 3), lambda i: (i, 0)),
            pl.BlockSpec((VI, VI), lambda i: (0, 0)),
        ],
        out_specs=[
            pl.BlockSpec((bn, TDST_W), lambda i: (i, 0)),
            pl.BlockSpec((bn, TSRC_W), lambda i: (i, 0)),
        ],
        out_shape=[
            jax.ShapeDtypeStruct((n, TDST_W), jnp.float32),
            jax.ShapeDtypeStruct((n, TSRC_W), jnp.float32),
        ],
    )(s, vflat, pos, wvec_t)


GCH = 80


def _gather_call(tdst, tsrc, src_idx, dst_idx, e):
    n_workers = 32
    per = e // n_workers
    assert per % GCH == 0
    niter = per // GCH
    npair = niter // 2
    tail = niter - 2 * npair
    mesh = plsc.VectorSubcoreMesh(core_axis_name="c", subcore_axis_name="s")

    @functools.partial(
        pl.kernel,
        mesh=mesh,
        out_type=[
            jax.ShapeDtypeStruct((e, TDST_W), jnp.float32),
            jax.ShapeDtypeStruct((e, TSRC_W), jnp.float32),
        ],
        scratch_types=[
            pltpu.VMEM((2, GCH), jnp.int32),
            pltpu.VMEM((2, GCH), jnp.int32),
            pltpu.VMEM((GCH, TDST_W), jnp.float32),
            pltpu.VMEM((GCH, TSRC_W), jnp.float32),
            pltpu.VMEM((GCH, TDST_W), jnp.float32),
            pltpu.VMEM((GCH, TSRC_W), jnp.float32),
            pltpu.SemaphoreType.DMA,
            pltpu.SemaphoreType.DMA,
            pltpu.SemaphoreType.DMA,
            pltpu.SemaphoreType.DMA,
        ],
    )
    def gather_k(tdst_hbm, tsrc_hbm, src_hbm, dst_hbm,
                 otd_hbm, ots_hbm,
                 dsti_v, srci_v, btd0, bts0, btd1, bts1,
                 gsem0, gsem1, wsem0, wsem1):
        wid = lax.axis_index("s") * 2 + lax.axis_index("c")
        tile_base = wid * per

        def wait_writes(btd, bts, wsem):
            pltpu.make_async_copy(btd, otd_hbm.at[pl.ds(tile_base, GCH)], wsem).wait()
            pltpu.make_async_copy(bts, ots_hbm.at[pl.ds(tile_base, GCH)], wsem).wait()

        def body(j, carry):
            a = tile_base + (2 * j) * GCH
            b = a + GCH

            @pl.when(j > 0)
            def _():
                wait_writes(btd0, bts0, wsem0)

            pltpu.sync_copy(dst_hbm.at[pl.ds(a, GCH)], dsti_v.at[0])
            pltpu.sync_copy(src_hbm.at[pl.ds(a, GCH)], srci_v.at[0])
            g1 = pltpu.async_copy(tdst_hbm.at[dsti_v.at[0]], btd0, gsem0)
            g2 = pltpu.async_copy(tsrc_hbm.at[srci_v.at[0]], bts0, gsem0)

            @pl.when(j > 0)
            def _():
                wait_writes(btd1, bts1, wsem1)

            pltpu.sync_copy(dst_hbm.at[pl.ds(b, GCH)], dsti_v.at[1])
            pltpu.sync_copy(src_hbm.at[pl.ds(b, GCH)], srci_v.at[1])
            g3 = pltpu.async_copy(tdst_hbm.at[dsti_v.at[1]], btd1, gsem1)
            g4 = pltpu.async_copy(tsrc_hbm.at[srci_v.at[1]], bts1, gsem1)

            g1.wait()
            g2.wait()
            pltpu.async_copy(btd0, otd_hbm.at[pl.ds(a, GCH)], wsem0)
            pltpu.async_copy(bts0, ots_hbm.at[pl.ds(a, GCH)], wsem0)
            g3.wait()
            g4.wait()
            pltpu.async_copy(btd1, otd_hbm.at[pl.ds(b, GCH)], wsem1)
            pltpu.async_copy(bts1, ots_hbm.at[pl.ds(b, GCH)], wsem1)
            return carry

        lax.fori_loop(0, npair, body, 0)
        wait_writes(btd0, bts0, wsem0)
        wait_writes(btd1, bts1, wsem1)
        if tail:
            t = tile_base + (2 * npair) * GCH
            pltpu.sync_copy(dst_hbm.at[pl.ds(t, GCH)], dsti_v.at[0])
            pltpu.sync_copy(src_hbm.at[pl.ds(t, GCH)], srci_v.at[0])
            g1 = pltpu.async_copy(tdst_hbm.at[dsti_v.at[0]], btd0, gsem0)
            g2 = pltpu.async_copy(tsrc_hbm.at[srci_v.at[0]], bts0, gsem0)
            g1.wait()
            g2.wait()
            pltpu.sync_copy(btd0, otd_hbm.at[pl.ds(t, GCH)])
            pltpu.sync_copy(bts0, ots_hbm.at[pl.ds(t, GCH)])

    return gather_k(tdst, tsrc, src_idx, dst_idx)


def _edge_body(tdst_ref, tsrc_ref, w1_t_ref, b1_ref, w2_t_ref,
               b2_ref, ms_ref, mvd_ref):
    tdst = tdst_ref[...]
    tsrc = tsrc_ref[...]
    s_j = tsrc[:, :SI]
    b = tdst.shape[0]

    s_i = tdst[:, :SI]
    pos_i = tdst[:, SI:SI + 3]
    pos_j = tsrc[:, SI + 3 * VI:SI + 3 * VI + 3]
    rel = pos_i - pos_j
    d2 = jnp.sum(rel * rel, axis=1, keepdims=True)
    d = jnp.sqrt(d2 + 1e-12)
    inv_d = 1.0 / d
    r_unit = rel * inv_d

    ks = lax.broadcasted_iota(jnp.int32, (1, K), 1).astype(jnp.float32) + 1.0
    t = (d * (1.0 / CUTOFF)) * ks
    r = jnp.round(t)
    w = t - r
    half_par = r * 0.5
    sign = 1.0 - 4.0 * (half_par - jnp.floor(half_par))
    de = sign * jnp.sin(np.pi * w) * inv_d * np.sqrt(2.0 / CUTOFF)
    rs = d * (1.0 / CUTOFF)
    rs2 = rs * rs
    rs3 = rs2 * rs
    rs6 = rs3 * rs3
    rs7 = rs6 * rs
    rs8 = rs7 * rs
    dc = (1.0 - 28.0 * rs6 + 48.0 * rs7 - 21.0 * rs8) * (rs < 1.0)
    de = de * dc

    w1 = w1_t_ref[...]
    hpre = (jnp.dot(s_i, w1[:SI], preferred_element_type=jnp.float32)
            + jnp.dot(s_j, w1[SI:2 * SI], preferred_element_type=jnp.float32)
            + jnp.dot(de, w1[2 * SI:], preferred_element_type=jnp.float32)
            + b1_ref[...])
    h = _silu(hpre)
    a = jnp.dot(h, w2_t_ref[...], preferred_element_type=jnp.float32) + b2_ref[...]
    a_s = a[:, :SI]
    w0 = a[:, SI:SI + VI]
    w1g = a[:, SI + VI:SI + 2 * VI]
    w2g = a[:, SI + 2 * VI:]

    ms_ref[...] = a_s * s_j

    vj = [tsrc[:, SI + c * VI:SI + (c + 1) * VI] for c in range(3)]
    r = [r_unit[:, c:c + 1] for c in range(3)]
    cross = [vj[1] * r[2] - vj[2] * r[1],
             vj[2] * r[0] - vj[0] * r[2],
             vj[0] * r[1] - vj[1] * r[0]]
    mv = [w0 * vj[c] + w1g * r[c] + w2g * cross[c] for c in range(3)]
    ones = jnp.ones((b, 1), jnp.float32)
    zpad = jnp.zeros((b, VI - 1), jnp.float32)
    mvd_ref[...] = jnp.concatenate(mv + [ones, zpad], axis=1)


def _edge_call(tdg, tsg, w1_t, b1, w2_t, b2, e):
    be = 1280
    assert e % be == 0
    grid = (e // be,)
    return pl.pallas_call(
        _edge_body,
        grid=grid,
        in_specs=[
            pl.BlockSpec((be, TDST_W), lambda i: (i, 0)),
            pl.BlockSpec((be, TSRC_W), lambda i: (i, 0)),
            pl.BlockSpec((2 * SI + K, SI), lambda i: (0, 0)),
            pl.BlockSpec((1, SI), lambda i: (0, 0)),
            pl.BlockSpec((SI, SI + 3 * VI), lambda i: (0, 0)),
            pl.BlockSpec((1, SI + 3 * VI), lambda i: (0, 0)),
        ],
        out_specs=[
            pl.BlockSpec((be, SI), lambda i: (i, 0)),
            pl.BlockSpec((be, SI), lambda i: (i, 0)),
        ],
        out_shape=[
            jax.ShapeDtypeStruct((e, SI), jnp.float32),
            jax.ShapeDtypeStruct((e, SI), jnp.float32),
        ],
    )(tdg, tsg, w1_t, b1, w2_t, b2)


SCH = 80


def _scatter_call(ms, mvd, dst_idx, zeros_n, n, e):
    n_tiles = 16
    per = e // n_tiles
    assert per % (2 * SCH) == 0
    npair = per // (2 * SCH)
    rows_per = (n // n_tiles) & ~7
    tail_base = n_tiles * rows_per
    tail = n - tail_base
    mesh = plsc.VectorSubcoreMesh(core_axis_name="c", subcore_axis_name="s")

    @functools.partial(
        pl.kernel,
        mesh=mesh,
        out_type=[
            jax.ShapeDtypeStruct((n, SI), jnp.float32),
            jax.ShapeDtypeStruct((n, SI), jnp.float32),
        ],
        scratch_types=[
            pltpu.VMEM((2, SCH), jnp.int32),
            pltpu.VMEM((SCH, SI), jnp.float32),
            pltpu.VMEM((SCH, SI), jnp.float32),
            pltpu.VMEM_SHARED((n, SI), jnp.float32),
            pltpu.SemaphoreType.DMA,
            pltpu.SemaphoreType.DMA,
            pltpu.SemaphoreType.DMA,
            pltpu.SemaphoreType.DMA,
        ],
    )
    def scatter_k(ms_hbm, mvd_hbm, dst_hbm, zeros_hbm, oms_hbm, omvd_hbm,
                  idx_v, row0, row1, acc_sh, rsem0, rsem1, ssem0, ssem1):
        cid = lax.axis_index("c")
        sid = lax.axis_index("s")
        pltpu.sync_copy(zeros_hbm.at[pl.ds(sid * rows_per, rows_per)],
                        acc_sh.at[pl.ds(sid * rows_per, rows_per)])

        @pl.when(sid == 0)
        def _():
            pltpu.sync_copy(zeros_hbm.at[pl.ds(tail_base, tail)],
                            acc_sh.at[pl.ds(tail_base, tail)])

        plsc.subcore_barrier()

        def run(src_hbm):
            def body(j, carry):
                a = sid * per + (2 * j) * SCH
                b = a + SCH

                @pl.when(j > 0)
                def _():
                    pltpu.make_async_copy(
                        row0, acc_sh.at[idx_v.at[0]], ssem0).wait()

                pltpu.sync_copy(dst_hbm.at[pl.ds(a, SCH)], idx_v.at[0])
                r0 = pltpu.async_copy(src_hbm.at[pl.ds(a, SCH)], row0, rsem0)

                @pl.when(j > 0)
                def _():
                    pltpu.make_async_copy(
                        row1, acc_sh.at[idx_v.at[1]], ssem1).wait()

                pltpu.sync_copy(dst_hbm.at[pl.ds(b, SCH)], idx_v.at[1])
                r1 = pltpu.async_copy(src_hbm.at[pl.ds(b, SCH)], row1, rsem1)

                r0.wait()
                pltpu.async_copy(row0, acc_sh.at[idx_v.at[0]], ssem0, add=True)
                r1.wait()
                pltpu.async_copy(row1, acc_sh.at[idx_v.at[1]], ssem1, add=True)
                return carry

            lax.fori_loop(0, npair, body, 0)
            pltpu.make_async_copy(row0, acc_sh.at[idx_v.at[0]], ssem0).wait()
            pltpu.make_async_copy(row1, acc_sh.at[idx_v.at[1]], ssem1).wait()

        @pl.when(cid == 0)
        def _():
            run(ms_hbm)

        @pl.when(cid == 1)
        def _():
            run(mvd_hbm)

        plsc.subcore_barrier()

        @pl.when(cid == 0)
        def _():
            pltpu.sync_copy(acc_sh.at[pl.ds(sid * rows_per, rows_per)],
                            oms_hbm.at[pl.ds(sid * rows_per, rows_per)])

            @pl.when(sid == 0)
            def _():
                pltpu.sync_copy(acc_sh.at[pl.ds(tail_base, tail)],
                                oms_hbm.at[pl.ds(tail_base, tail)])

        @pl.when(cid == 1)
        def _():
            pltpu.sync_copy(acc_sh.at[pl.ds(sid * rows_per, rows_per)],
                            omvd_hbm.at[pl.ds(sid * rows_per, rows_per)])

            @pl.when(sid == 0)
            def _():
                pltpu.sync_copy(acc_sh.at[pl.ds(tail_base, tail)],
                                omvd_hbm.at[pl.ds(tail_base, tail)])

    return scatter_k(ms, mvd, dst_idx, zeros_n)


def _node_body(msagg_ref, mvd_ref, s_ref, v_ref, wv0_t_ref, ws1_t_ref,
               bs1_ref, ws2_t_ref, bs2_ref, wv1_t_ref, sout_ref, vout_ref):
    ms_agg = msagg_ref[...]
    mvd = mvd_ref[...]
    s = s_ref[...]
    vflat = v_ref[...]

    deg = mvd[:, 3 * VI:3 * VI + 1]
    inv_deg = 1.0 / jnp.maximum(deg, 1.0)
    wv0 = wv0_t_ref[...]
    vv = [jnp.dot(mvd[:, c * VI:(c + 1) * VI] * inv_deg, wv0,
                  preferred_element_type=jnp.float32) for c in range(3)]
    vn2 = sum(vv[c][:, :VI] * vv[c][:, :VI] for c in range(3))
    vnorm = jnp.sqrt(jnp.maximum(vn2, 1e-6))
    scat = jnp.concatenate([ms_agg, vnorm], axis=1)
    hh = _silu(jnp.dot(scat, ws1_t_ref[...], preferred_element_type=jnp.float32)
               + bs1_ref[...])
    o = jnp.dot(hh, ws2_t_ref[...], preferred_element_type=jnp.float32) + bs2_ref[...]
    gate = o[:, :VI]
    s_upd = o[:, VI:]
    wv1 = wv1_t_ref[...]
    vupd = [jnp.dot(gate * vv[c][:, VI:], wv1,
                    preferred_element_type=jnp.float32) for c in range(3)]
    sout_ref[...] = s + s_upd
    vout_ref[...] = vflat + jnp.concatenate(vupd, axis=1)


def _node_call(ms_agg, mvd_agg, s, vflat, wv0_t, ws1_t, bs1, ws2_t, bs2,
               wv1_t, n):
    bn = 1000
    grid = (n // bn,)
    return pl.pallas_call(
        _node_body,
        grid=grid,
        in_specs=[
            pl.BlockSpec((bn, SI), lambda i: (i, 0)),
            pl.BlockSpec((bn, SI), lambda i: (i, 0)),
            pl.BlockSpec((bn, SI), lambda i: (i, 0)),
            pl.BlockSpec((bn, 3 * VI), lambda i: (i, 0)),
            pl.BlockSpec((VI, 2 * VI), lambda i: (0, 0)),
            pl.BlockSpec((VI + SI, SI), lambda i: (0, 0)),
            pl.BlockSpec((1, SI), lambda i: (0, 0)),
            pl.BlockSpec((SI, VI + SI), lambda i: (0, 0)),
            pl.BlockSpec((1, VI + SI), lambda i: (0, 0)),
            pl.BlockSpec((VI, VI), lambda i: (0, 0)),
        ],
        out_specs=[
            pl.BlockSpec((bn, SI), lambda i: (i, 0)),
            pl.BlockSpec((bn, 3 * VI), lambda i: (i, 0)),
        ],
        out_shape=[
            jax.ShapeDtypeStruct((n, SI), jnp.float32),
            jax.ShapeDtypeStruct((n, 3 * VI), jnp.float32),
        ],
    )(ms_agg, mvd_agg, s, vflat, wv0_t, ws1_t, bs1, ws2_t, bs2, wv1_t)


def kernel(s, v, pos, edge_index, W_edge1, b_edge1, W_edge2, b_edge2,
           W_vec, W_v0, W_s1, b_s1, W_s2, b_s2, W_v1):
    n = s.shape[0]
    e = edge_index.shape[1]
    vflat = v.reshape(n, 3 * VI)
    src = edge_index[0]
    dst = edge_index[1]

    tdst, tsrc = _prep_call(s, vflat, pos, W_vec.T, n)

    tdg, tsg = _gather_call(tdst, tsrc, src, dst, e)

    ms, mvd = _edge_call(tdg, tsg, W_edge1.T, b_edge1[None, :],
                         W_edge2.T, b_edge2[None, :], e)

    zeros_n = jnp.zeros((n, SI), jnp.float32)
    ms_agg, mvd_agg = _scatter_call(ms, mvd, dst, zeros_n, n, e)

    s_out, v_out = _node_call(ms_agg, mvd_agg, s, vflat, W_v0.T, W_s1.T,
                              b_s1[None, :], W_s2.T, b_s2[None, :], W_v1.T, n)
    return (s_out, v_out.reshape(n, 3, VI))
